# baseline (speedup 1.0000x reference)
import functools

import jax
import jax.numpy as jnp
from jax import lax
from jax.experimental import pallas as pl
from jax.experimental.pallas import tpu as pltpu
from jax.experimental.pallas import tpu_sc as plsc

BATCH = 16384
DIM = 16
NS = 16
B_PER_W = BATCH // NS
N_CHUNK = B_PER_W // 128


def _lane_shuffle(v, perm):
    return lax.gather(
        v, perm[:, None],
        dimension_numbers=lax.GatherDimensionNumbers(
            offset_dims=(), collapsed_slice_dims=(0,), start_index_map=(0,)),
        slice_sizes=(1,),
        mode=lax.GatherScatterMode.PROMISE_IN_BOUNDS)


def _rsqrt_newton(x):
    i = lax.bitcast_convert_type(x, jnp.int32)
    i = jnp.int32(0x5F3759DF) - (i >> 1)
    y = lax.bitcast_convert_type(i, jnp.float32)
    for _ in range(4):
        y = y * (jnp.float32(1.5) - jnp.float32(0.5) * x * y * y)
    return y


def _dmf_body(uidx_hbm, iidx_hbm, utab_hbm, itab_hbm, out_hbm,
              uidx_v, iidx_v, urows_v, irows_v, yv, part_v, allp_v,
              shared, sem):
    wid = lax.axis_index("s")
    base = wid * B_PER_W

    pltpu.sync_copy(uidx_hbm.at[wid], uidx_v)
    pltpu.sync_copy(iidx_hbm.at[wid], iidx_v)

    copies = []
    for j in range(N_CHUNK):
        copies.append(pltpu.async_copy(
            utab_hbm.at[uidx_v.at[j]], urows_v.at[pl.ds(j * 128, 128)], sem))
        copies.append(pltpu.async_copy(
            itab_hbm.at[iidx_v.at[j]], irows_v.at[pl.ds(j * 128, 128)], sem))
    for c in copies:
        c.wait()

    def acc(i, carry):
        su, sv = carry
        u = urows_v[i, :]
        v = irows_v[i, :]
        return su + u * u, sv + v * v

    zeros = jnp.zeros((16,), jnp.float32)
    su, sv = lax.fori_loop(0, B_PER_W, acc, (zeros, zeros))

    part_v[0, :] = su
    part_v[1, :] = sv
    pltpu.sync_copy(part_v, shared.at[wid])
    plsc.subcore_barrier()
    pltpu.sync_copy(shared, allp_v)

    su_tot = allp_v[0, 0, :]
    sv_tot = allp_v[0, 1, :]
    for j in range(1, NS):
        su_tot = su_tot + allp_v[j, 0, :]
        sv_tot = sv_tot + allp_v[j, 1, :]
    lanes = lax.iota(jnp.int32, 16)
    for k in (1, 2, 4, 8):
        perm = lanes ^ k
        su_tot = su_tot + _lane_shuffle(su_tot, perm)
        sv_tot = sv_tot + _lane_shuffle(sv_tot, perm)
    inv = _rsqrt_newton(su_tot * sv_tot)

    def scale(i, _):
        yv[i, :] = urows_v[i, :] * irows_v[i, :] * inv
        return 0

    lax.fori_loop(0, B_PER_W, scale, 0)
    pltpu.sync_copy(yv, out_hbm.at[pl.ds(base, B_PER_W)])


@jax.jit
def kernel(user_indices, item_indices, user_table, item_table):
    uidx = user_indices.astype(jnp.int32).reshape(NS, N_CHUNK, 128)
    iidx = item_indices.astype(jnp.int32).reshape(NS, N_CHUNK, 128)
    mesh = plsc.VectorSubcoreMesh(
        core_axis_name="c", subcore_axis_name="s", num_cores=1,
        num_subcores=NS)
    run = pl.kernel(
        _dmf_body,
        out_type=jax.ShapeDtypeStruct((BATCH, DIM), jnp.float32),
        mesh=mesh,
        scratch_types=[
            pltpu.VMEM((N_CHUNK, 128), jnp.int32),
            pltpu.VMEM((N_CHUNK, 128), jnp.int32),
            pltpu.VMEM((B_PER_W, DIM), jnp.float32),
            pltpu.VMEM((B_PER_W, DIM), jnp.float32),
            pltpu.VMEM((B_PER_W, DIM), jnp.float32),
            pltpu.VMEM((2, 16), jnp.float32),
            pltpu.VMEM((NS, 2, 16), jnp.float32),
            pltpu.VMEM_SHARED((NS, 2, 16), jnp.float32),
            pltpu.SemaphoreType.DMA,
        ],
        compiler_params=pltpu.CompilerParams(use_tc_tiling_on_sc=False),
    )
    return run(uidx, iidx, user_table, item_table)


# speedup vs baseline: 3.5412x; 3.5412x over previous
import jax
import jax.numpy as jnp
import numpy as np
from jax import lax
from jax.experimental import pallas as pl
from jax.experimental.pallas import tpu as pltpu
from jax.experimental.pallas import tpu_sc as plsc

NROWS = 1_000_000
BATCH = 16384
DIM = 16
NW = 32
B_PER_W = BATCH // NW
GROUP = 16
NGROUPS = B_PER_W // GROUP
NTILES = NROWS // 128
LAST_BASE = NROWS - 128
SAFE_LIMIT = NTILES * 128


def _rsqrt_newton(x):
    i = lax.bitcast_convert_type(x, jnp.int32)
    i = jnp.int32(0x5F3759DF) - (i >> 1)
    y = lax.bitcast_convert_type(i, jnp.float32)
    for _ in range(4):
        y = y * (jnp.float32(1.5) - jnp.float32(0.5) * x * y * y)
    return y


def _lane_pick(vec, lane_vec):
    return lax.gather(
        vec, lane_vec[:, None],
        dimension_numbers=lax.GatherDimensionNumbers(
            offset_dims=(), collapsed_slice_dims=(0,), start_index_map=(0,)),
        slice_sizes=(1,),
        mode=lax.GatherScatterMode.PROMISE_IN_BOUNDS)


_J16 = tuple(range(DIM))


def _extract_row(slot, last_v, safe_f, chunk, chunk_last, lane_vec):
    vals = []
    for j in _J16:
        a = slot[j, pl.ds(chunk, 16)]
        b = last_v[j, pl.ds(chunk_last, 16)]
        ch = b + (a - b) * safe_f
        vals.append(_lane_pick(ch, lane_vec))
    return vals


def _gather_body(uidx_hbm, iidx_hbm, utab_hbm, itab_hbm, ulast_hbm,
                 ilast_hbm, y_hbm, part_hbm,
                 uidx_v, iidx_v, slot_u, slot_i, last_u, last_i, yv,
                 part_v, sem):
    wid = lax.axis_index("s") * 2 + lax.axis_index("c")
    base = wid * B_PER_W

    pltpu.sync_copy(uidx_hbm.at[pl.ds(base, B_PER_W)], uidx_v)
    pltpu.sync_copy(iidx_hbm.at[pl.ds(base, B_PER_W)], iidx_v)
    pltpu.sync_copy(ulast_hbm, last_u)
    pltpu.sync_copy(ilast_hbm, last_i)

    iota16 = lax.iota(jnp.int32, 16)
    zeros = jnp.zeros((16,), jnp.float32)
    onehot = [
        (1 - jnp.minimum(jnp.abs(iota16 - j), 1)).astype(jnp.float32)
        for j in _J16
    ]

    def group(g, carry):
        su, sv = carry
        uvec = uidx_v[pl.ds(g * GROUP, GROUP)]
        ivec = iidx_v[pl.ds(g * GROUP, GROUP)]

        descs = []
        rows = []
        for k in range(GROUP):
            ur = uvec[k]
            ir = ivec[k]
            uc = jnp.minimum(ur >> 7, NTILES - 1)
            ic = jnp.minimum(ir >> 7, NTILES - 1)
            ucol = pl.multiple_of(uc << 7, 128)
            icol = pl.multiple_of(ic << 7, 128)
            d = [
                pltpu.async_copy(
                    utab_hbm.at[pl.ds(0, 8), pl.ds(ucol, 128)],
                    slot_u.at[k, pl.ds(0, 8), :], sem.at[k]),
                pltpu.async_copy(
                    utab_hbm.at[pl.ds(8, 8), pl.ds(ucol, 128)],
                    slot_u.at[k, pl.ds(8, 8), :], sem.at[k]),
                pltpu.async_copy(
                    itab_hbm.at[pl.ds(0, 8), pl.ds(icol, 128)],
                    slot_i.at[k, pl.ds(0, 8), :], sem.at[k]),
                pltpu.async_copy(
                    itab_hbm.at[pl.ds(8, 8), pl.ds(icol, 128)],
                    slot_i.at[k, pl.ds(8, 8), :], sem.at[k]),
            ]
            descs.append(d)
            rows.append((ur, ir))

        for k in range(GROUP):
            for d in descs[k]:
                d.wait()
            ur, ir = rows[k]

            u_safe = ur < SAFE_LIMIT
            i_safe = ir < SAFE_LIMIT
            us = ur & 127
            is_ = ir & 127
            usl = (ur - LAST_BASE) & 127
            isl = (ir - LAST_BASE) & 127
            u_vals = _extract_row(
                slot_u.at[k], last_u,
                jnp.broadcast_to(
                    jnp.where(u_safe, jnp.float32(1.0), jnp.float32(0.0)),
                    (16,)),
                pl.multiple_of((us >> 4) << 4, 16),
                pl.multiple_of((usl >> 4) << 4, 16),
                jnp.broadcast_to(jnp.where(u_safe, us, usl) & 15, (16,)))
            i_vals = _extract_row(
                slot_i.at[k], last_i,
                jnp.broadcast_to(
                    jnp.where(i_safe, jnp.float32(1.0), jnp.float32(0.0)),
                    (16,)),
                pl.multiple_of((is_ >> 4) << 4, 16),
                pl.multiple_of((isl >> 4) << 4, 16),
                jnp.broadcast_to(jnp.where(i_safe, is_, isl) & 15, (16,)))

            p = zeros
            for j in _J16:
                u = u_vals[j]
                v = i_vals[j]
                su = su + u * u
                sv = sv + v * v
                p = p + (u * v) * onehot[j]
            yv[pl.ds((g * GROUP + k) * DIM, DIM)] = p
        return su, sv

    su, sv = lax.fori_loop(0, NGROUPS, group, (zeros, zeros))

    part_v[pl.ds(0, 16)] = su
    part_v[pl.ds(16, 16)] = sv
    pltpu.sync_copy(part_v, part_hbm.at[pl.ds(wid * 32, 32)])
    pltpu.sync_copy(yv, y_hbm.at[pl.ds(base * DIM, B_PER_W * DIM)])


def _scale_body(y_hbm, part_hbm, out_hbm, yv, part_v):
    wid = lax.axis_index("s") * 2 + lax.axis_index("c")
    base = wid * B_PER_W * DIM

    pltpu.sync_copy(part_hbm, part_v)
    pltpu.sync_copy(y_hbm.at[pl.ds(base, B_PER_W * DIM)], yv)

    su = part_v[pl.ds(0, 16)]
    sv = part_v[pl.ds(16, 16)]
    for w in range(1, NW):
        su = su + part_v[pl.ds(w * 32, 16)]
        sv = sv + part_v[pl.ds(w * 32 + 16, 16)]
    inv = _rsqrt_newton(su * sv)

    def scale(i, _):
        yv[pl.ds(i * 16, 16)] = yv[pl.ds(i * 16, 16)] * inv
        return 0

    lax.fori_loop(0, B_PER_W * DIM // 16, scale, 0)
    pltpu.sync_copy(yv, out_hbm.at[pl.ds(base, B_PER_W * DIM)])


@jax.jit
def kernel(user_indices, item_indices, user_table, item_table):
    uidx = user_indices.astype(jnp.int32)
    iidx = item_indices.astype(jnp.int32)
    ut = user_table.T
    it = item_table.T
    ulast = user_table[LAST_BASE:].T
    ilast = item_table[LAST_BASE:].T

    mesh = plsc.VectorSubcoreMesh(
        core_axis_name="c", subcore_axis_name="s", num_cores=2,
        num_subcores=16)
    gather = pl.kernel(
        _gather_body,
        out_type=(
            jax.ShapeDtypeStruct((BATCH * DIM,), jnp.float32),
            jax.ShapeDtypeStruct((NW * 32,), jnp.float32),
        ),
        mesh=mesh,
        scratch_types=[
            pltpu.VMEM((B_PER_W,), jnp.int32),
            pltpu.VMEM((B_PER_W,), jnp.int32),
            pltpu.VMEM((GROUP, 16, 128), jnp.float32),
            pltpu.VMEM((GROUP, 16, 128), jnp.float32),
            pltpu.VMEM((16, 128), jnp.float32),
            pltpu.VMEM((16, 128), jnp.float32),
            pltpu.VMEM((B_PER_W * DIM,), jnp.float32),
            pltpu.VMEM((32,), jnp.float32),
            pltpu.SemaphoreType.DMA((GROUP,)),
        ],
        compiler_params=pltpu.CompilerParams(use_tc_tiling_on_sc=True),
    )
    y_un, partials = gather(uidx, iidx, ut, it, ulast, ilast)

    scale = pl.kernel(
        _scale_body,
        out_type=jax.ShapeDtypeStruct((BATCH * DIM,), jnp.float32),
        mesh=mesh,
        scratch_types=[
            pltpu.VMEM((B_PER_W * DIM,), jnp.float32),
            pltpu.VMEM((NW * 32,), jnp.float32),
        ],
        compiler_params=pltpu.CompilerParams(use_tc_tiling_on_sc=True),
    )
    y = scale(y_un, partials)
    return y.reshape(BATCH, DIM)


# speedup vs baseline: 3.5892x; 1.0136x over previous
import jax
import jax.numpy as jnp
import numpy as np
from jax import lax
from jax.experimental import pallas as pl
from jax.experimental.pallas import tpu as pltpu
from jax.experimental.pallas import tpu_sc as plsc

NROWS = 1_000_000
BATCH = 16384
DIM = 16
NW = 32
B_PER_W = BATCH // NW
GROUP = 16
NGROUPS = B_PER_W // GROUP
NTILES = NROWS // 128
LAST_BASE = NROWS - 128
SAFE_LIMIT = NTILES * 128


def _rsqrt_newton(x):
    i = lax.bitcast_convert_type(x, jnp.int32)
    i = jnp.int32(0x5F3759DF) - (i >> 1)
    y = lax.bitcast_convert_type(i, jnp.float32)
    for _ in range(4):
        y = y * (jnp.float32(1.5) - jnp.float32(0.5) * x * y * y)
    return y


def _lane_pick(vec, lane_vec):
    return lax.gather(
        vec, lane_vec[:, None],
        dimension_numbers=lax.GatherDimensionNumbers(
            offset_dims=(), collapsed_slice_dims=(0,), start_index_map=(0,)),
        slice_sizes=(1,),
        mode=lax.GatherScatterMode.PROMISE_IN_BOUNDS)


_J16 = tuple(range(DIM))


def _extract_row(slot, last_v, safe_f, chunk, chunk_last, lane_vec):
    vals = []
    for j in _J16:
        a = slot[j, pl.ds(chunk, 16)]
        b = last_v[j, pl.ds(chunk_last, 16)]
        ch = b + (a - b) * safe_f
        vals.append(_lane_pick(ch, lane_vec))
    return vals


def _gather_body(uidx_hbm, iidx_hbm, utab_hbm, itab_hbm, ulast_hbm,
                 ilast_hbm, y_hbm, part_hbm,
                 uidx_v, iidx_v, slot_u, slot_i, last_u, last_i, yv,
                 part_v, sem):
    wid = lax.axis_index("s") * 2 + lax.axis_index("c")
    base = wid * B_PER_W

    pltpu.sync_copy(uidx_hbm.at[pl.ds(base, B_PER_W)], uidx_v)
    pltpu.sync_copy(iidx_hbm.at[pl.ds(base, B_PER_W)], iidx_v)
    pltpu.sync_copy(ulast_hbm, last_u)
    pltpu.sync_copy(ilast_hbm, last_i)

    iota16 = lax.iota(jnp.int32, 16)
    zeros = jnp.zeros((16,), jnp.float32)
    onehot = [
        (1 - jnp.minimum(jnp.abs(iota16 - j), 1)).astype(jnp.float32)
        for j in _J16
    ]

    def group(g, carry):
        su, sv = carry
        uvec = uidx_v[pl.ds(g * GROUP, GROUP)]
        ivec = iidx_v[pl.ds(g * GROUP, GROUP)]

        descs = []
        rows = []
        for k in range(GROUP):
            ur = uvec[k]
            ir = ivec[k]
            uc = jnp.minimum(ur >> 7, NTILES - 1)
            ic = jnp.minimum(ir >> 7, NTILES - 1)
            ucol = pl.multiple_of(uc << 7, 128)
            icol = pl.multiple_of(ic << 7, 128)
            d = [
                pltpu.async_copy(
                    utab_hbm.at[:, pl.ds(ucol, 128)],
                    slot_u.at[k], sem.at[k]),
                pltpu.async_copy(
                    itab_hbm.at[:, pl.ds(icol, 128)],
                    slot_i.at[k], sem.at[k]),
            ]
            descs.append(d)
            rows.append((ur, ir))

        for k in range(GROUP):
            for d in descs[k]:
                d.wait()
            ur, ir = rows[k]

            u_safe = ur < SAFE_LIMIT
            i_safe = ir < SAFE_LIMIT
            us = ur & 127
            is_ = ir & 127
            usl = (ur - LAST_BASE) & 127
            isl = (ir - LAST_BASE) & 127
            u_vals = _extract_row(
                slot_u.at[k], last_u,
                jnp.broadcast_to(
                    jnp.where(u_safe, jnp.float32(1.0), jnp.float32(0.0)),
                    (16,)),
                pl.multiple_of((us >> 4) << 4, 16),
                pl.multiple_of((usl >> 4) << 4, 16),
                jnp.broadcast_to(jnp.where(u_safe, us, usl) & 15, (16,)))
            i_vals = _extract_row(
                slot_i.at[k], last_i,
                jnp.broadcast_to(
                    jnp.where(i_safe, jnp.float32(1.0), jnp.float32(0.0)),
                    (16,)),
                pl.multiple_of((is_ >> 4) << 4, 16),
                pl.multiple_of((isl >> 4) << 4, 16),
                jnp.broadcast_to(jnp.where(i_safe, is_, isl) & 15, (16,)))

            p = zeros
            for j in _J16:
                u = u_vals[j]
                v = i_vals[j]
                su = su + u * u
                sv = sv + v * v
                p = p + (u * v) * onehot[j]
            yv[pl.ds((g * GROUP + k) * DIM, DIM)] = p
        return su, sv

    su, sv = lax.fori_loop(0, NGROUPS, group, (zeros, zeros))

    part_v[pl.ds(0, 16)] = su
    part_v[pl.ds(16, 16)] = sv
    pltpu.sync_copy(part_v, part_hbm.at[pl.ds(wid * 32, 32)])
    pltpu.sync_copy(yv, y_hbm.at[pl.ds(base * DIM, B_PER_W * DIM)])


def _scale_body(y_hbm, part_hbm, out_hbm, yv, part_v):
    wid = lax.axis_index("s") * 2 + lax.axis_index("c")
    base = wid * B_PER_W * DIM

    pltpu.sync_copy(part_hbm, part_v)
    pltpu.sync_copy(y_hbm.at[pl.ds(base, B_PER_W * DIM)], yv)

    su = part_v[pl.ds(0, 16)]
    sv = part_v[pl.ds(16, 16)]
    for w in range(1, NW):
        su = su + part_v[pl.ds(w * 32, 16)]
        sv = sv + part_v[pl.ds(w * 32 + 16, 16)]
    inv = _rsqrt_newton(su * sv)

    def scale(i, _):
        yv[pl.ds(i * 16, 16)] = yv[pl.ds(i * 16, 16)] * inv
        return 0

    lax.fori_loop(0, B_PER_W * DIM // 16, scale, 0)
    pltpu.sync_copy(yv, out_hbm.at[pl.ds(base, B_PER_W * DIM)])


@jax.jit
def kernel(user_indices, item_indices, user_table, item_table):
    uidx = user_indices.astype(jnp.int32)
    iidx = item_indices.astype(jnp.int32)
    ut = user_table.T
    it = item_table.T
    ulast = user_table[LAST_BASE:].T
    ilast = item_table[LAST_BASE:].T

    mesh = plsc.VectorSubcoreMesh(
        core_axis_name="c", subcore_axis_name="s", num_cores=2,
        num_subcores=16)
    gather = pl.kernel(
        _gather_body,
        out_type=(
            jax.ShapeDtypeStruct((BATCH * DIM,), jnp.float32),
            jax.ShapeDtypeStruct((NW * 32,), jnp.float32),
        ),
        mesh=mesh,
        scratch_types=[
            pltpu.VMEM((B_PER_W,), jnp.int32),
            pltpu.VMEM((B_PER_W,), jnp.int32),
            pltpu.VMEM((GROUP, 16, 128), jnp.float32),
            pltpu.VMEM((GROUP, 16, 128), jnp.float32),
            pltpu.VMEM((16, 128), jnp.float32),
            pltpu.VMEM((16, 128), jnp.float32),
            pltpu.VMEM((B_PER_W * DIM,), jnp.float32),
            pltpu.VMEM((32,), jnp.float32),
            pltpu.SemaphoreType.DMA((GROUP,)),
        ],
        compiler_params=pltpu.CompilerParams(use_tc_tiling_on_sc=True),
    )
    y_un, partials = gather(uidx, iidx, ut, it, ulast, ilast)

    scale = pl.kernel(
        _scale_body,
        out_type=jax.ShapeDtypeStruct((BATCH * DIM,), jnp.float32),
        mesh=mesh,
        scratch_types=[
            pltpu.VMEM((B_PER_W * DIM,), jnp.float32),
            pltpu.VMEM((NW * 32,), jnp.float32),
        ],
        compiler_params=pltpu.CompilerParams(use_tc_tiling_on_sc=True),
    )
    y = scale(y_un, partials)
    return y.reshape(BATCH, DIM)


# speedup vs baseline: 5.0484x; 1.4066x over previous
import jax
import jax.numpy as jnp
import numpy as np
from jax import lax
from jax.experimental import pallas as pl
from jax.experimental.pallas import tpu as pltpu
from jax.experimental.pallas import tpu_sc as plsc

NROWS = 1_000_000
BATCH = 16384
DIM = 16
NW = 32
B_PER_W = BATCH // NW
GROUP = 16
NGROUPS = B_PER_W // GROUP
NTILES = NROWS // 128
LAST_BASE = NROWS - 128
SAFE_LIMIT = NTILES * 128


def _rsqrt_newton(x):
    i = lax.bitcast_convert_type(x, jnp.int32)
    i = jnp.int32(0x5F3759DF) - (i >> 1)
    y = lax.bitcast_convert_type(i, jnp.float32)
    for _ in range(4):
        y = y * (jnp.float32(1.5) - jnp.float32(0.5) * x * y * y)
    return y


def _lane_pick(vec, lane_vec):
    return lax.gather(
        vec, lane_vec[:, None],
        dimension_numbers=lax.GatherDimensionNumbers(
            offset_dims=(), collapsed_slice_dims=(0,), start_index_map=(0,)),
        slice_sizes=(1,),
        mode=lax.GatherScatterMode.PROMISE_IN_BOUNDS)


_J16 = tuple(range(DIM))


def _extract_row(slot, last_v, safe_f, chunk, chunk_last, lane_vec):
    vals = []
    for j in _J16:
        a = slot[j, pl.ds(chunk, 16)]
        b = last_v[j, pl.ds(chunk_last, 16)]
        ch = b + (a - b) * safe_f
        vals.append(_lane_pick(ch, lane_vec))
    return vals


def _gather_body(uidx_hbm, iidx_hbm, utab_hbm, itab_hbm, ulast_hbm,
                 ilast_hbm, y_hbm, part_hbm,
                 uidx_v, iidx_v, slot_u, slot_i, last_u, last_i, yv,
                 part_v, sem):
    wid = lax.axis_index("s") * 2 + lax.axis_index("c")
    base = wid * B_PER_W

    pltpu.sync_copy(uidx_hbm.at[pl.ds(base, B_PER_W)], uidx_v)
    pltpu.sync_copy(iidx_hbm.at[pl.ds(base, B_PER_W)], iidx_v)
    pltpu.sync_copy(ulast_hbm, last_u)
    pltpu.sync_copy(ilast_hbm, last_i)

    iota16 = lax.iota(jnp.int32, 16)
    zeros = jnp.zeros((16,), jnp.float32)
    onehot = [
        (1 - jnp.minimum(jnp.abs(iota16 - j), 1)).astype(jnp.float32)
        for j in _J16
    ]

    def fire(uvec, ivec, k):
        ur = uvec[k]
        ir = ivec[k]
        uc = jnp.minimum(ur >> 7, NTILES - 1)
        ic = jnp.minimum(ir >> 7, NTILES - 1)
        ucol = pl.multiple_of(uc << 7, 128)
        icol = pl.multiple_of(ic << 7, 128)
        pltpu.async_copy(utab_hbm.at[:, pl.ds(ucol, 128)],
                         slot_u.at[k], sem.at[k])
        pltpu.async_copy(itab_hbm.at[:, pl.ds(icol, 128)],
                         slot_i.at[k], sem.at[k])

    uvec0 = uidx_v[pl.ds(0, GROUP)]
    ivec0 = iidx_v[pl.ds(0, GROUP)]
    for k in range(GROUP):
        fire(uvec0, ivec0, k)

    def group(g, carry):
        su, sv = carry
        uvec = uidx_v[pl.ds(g * GROUP, GROUP)]
        ivec = iidx_v[pl.ds(g * GROUP, GROUP)]
        gn = jnp.minimum(g + 1, NGROUPS - 1)
        nuvec = uidx_v[pl.ds(gn * GROUP, GROUP)]
        nivec = iidx_v[pl.ds(gn * GROUP, GROUP)]

        for k in range(GROUP):
            pltpu.make_async_copy(utab_hbm.at[:, pl.ds(0, 128)],
                                  slot_u.at[k], sem.at[k]).wait()
            pltpu.make_async_copy(itab_hbm.at[:, pl.ds(0, 128)],
                                  slot_i.at[k], sem.at[k]).wait()
            ur = uvec[k]
            ir = ivec[k]

            u_safe = ur < SAFE_LIMIT
            i_safe = ir < SAFE_LIMIT
            us = ur & 127
            is_ = ir & 127
            usl = (ur - LAST_BASE) & 127
            isl = (ir - LAST_BASE) & 127
            u_vals = _extract_row(
                slot_u.at[k], last_u,
                jnp.broadcast_to(
                    jnp.where(u_safe, jnp.float32(1.0), jnp.float32(0.0)),
                    (16,)),
                pl.multiple_of((us >> 4) << 4, 16),
                pl.multiple_of((usl >> 4) << 4, 16),
                jnp.broadcast_to(jnp.where(u_safe, us, usl) & 15, (16,)))
            i_vals = _extract_row(
                slot_i.at[k], last_i,
                jnp.broadcast_to(
                    jnp.where(i_safe, jnp.float32(1.0), jnp.float32(0.0)),
                    (16,)),
                pl.multiple_of((is_ >> 4) << 4, 16),
                pl.multiple_of((isl >> 4) << 4, 16),
                jnp.broadcast_to(jnp.where(i_safe, is_, isl) & 15, (16,)))

            p = zeros
            for j in _J16:
                u = u_vals[j]
                v = i_vals[j]
                su = su + u * u
                sv = sv + v * v
                p = p + (u * v) * onehot[j]
            yv[pl.ds((g * GROUP + k) * DIM, DIM)] = p

            @pl.when(g + 1 < NGROUPS)
            def _():
                fire(nuvec, nivec, k)

        return su, sv

    su, sv = lax.fori_loop(0, NGROUPS, group, (zeros, zeros))

    part_v[pl.ds(0, 16)] = su
    part_v[pl.ds(16, 16)] = sv
    pltpu.sync_copy(part_v, part_hbm.at[pl.ds(wid * 32, 32)])
    pltpu.sync_copy(yv, y_hbm.at[pl.ds(base * DIM, B_PER_W * DIM)])


def _scale_body(y_hbm, part_hbm, out_hbm, yv, part_v):
    wid = lax.axis_index("s") * 2 + lax.axis_index("c")
    base = wid * B_PER_W * DIM

    pltpu.sync_copy(part_hbm, part_v)
    pltpu.sync_copy(y_hbm.at[pl.ds(base, B_PER_W * DIM)], yv)

    su = part_v[pl.ds(0, 16)]
    sv = part_v[pl.ds(16, 16)]
    for w in range(1, NW):
        su = su + part_v[pl.ds(w * 32, 16)]
        sv = sv + part_v[pl.ds(w * 32 + 16, 16)]
    inv = _rsqrt_newton(su * sv)

    def scale(i, _):
        yv[pl.ds(i * 16, 16)] = yv[pl.ds(i * 16, 16)] * inv
        return 0

    lax.fori_loop(0, B_PER_W * DIM // 16, scale, 0)
    pltpu.sync_copy(yv, out_hbm.at[pl.ds(base, B_PER_W * DIM)])


@jax.jit
def kernel(user_indices, item_indices, user_table, item_table):
    uidx = user_indices.astype(jnp.int32)
    iidx = item_indices.astype(jnp.int32)
    ut = user_table.T
    it = item_table.T
    ulast = user_table[LAST_BASE:].T
    ilast = item_table[LAST_BASE:].T

    mesh = plsc.VectorSubcoreMesh(
        core_axis_name="c", subcore_axis_name="s", num_cores=2,
        num_subcores=16)
    gather = pl.kernel(
        _gather_body,
        out_type=(
            jax.ShapeDtypeStruct((BATCH * DIM,), jnp.float32),
            jax.ShapeDtypeStruct((NW * 32,), jnp.float32),
        ),
        mesh=mesh,
        scratch_types=[
            pltpu.VMEM((B_PER_W,), jnp.int32),
            pltpu.VMEM((B_PER_W,), jnp.int32),
            pltpu.VMEM((GROUP, 16, 128), jnp.float32),
            pltpu.VMEM((GROUP, 16, 128), jnp.float32),
            pltpu.VMEM((16, 128), jnp.float32),
            pltpu.VMEM((16, 128), jnp.float32),
            pltpu.VMEM((B_PER_W * DIM,), jnp.float32),
            pltpu.VMEM((32,), jnp.float32),
            pltpu.SemaphoreType.DMA((GROUP,)),
        ],
        compiler_params=pltpu.CompilerParams(use_tc_tiling_on_sc=True),
    )
    y_un, partials = gather(uidx, iidx, ut, it, ulast, ilast)

    scale = pl.kernel(
        _scale_body,
        out_type=jax.ShapeDtypeStruct((BATCH * DIM,), jnp.float32),
        mesh=mesh,
        scratch_types=[
            pltpu.VMEM((B_PER_W * DIM,), jnp.float32),
            pltpu.VMEM((NW * 32,), jnp.float32),
        ],
        compiler_params=pltpu.CompilerParams(use_tc_tiling_on_sc=True),
    )
    y = scale(y_un, partials)
    return y.reshape(BATCH, DIM)


# speedup vs baseline: 5.1594x; 1.0220x over previous
import jax
import jax.numpy as jnp
import numpy as np
from jax import lax
from jax.experimental import pallas as pl
from jax.experimental.pallas import tpu as pltpu
from jax.experimental.pallas import tpu_sc as plsc

NROWS = 1_000_000
BATCH = 16384
DIM = 16
NW = 32
B_PER_W = BATCH // NW
GROUP = 16
NGROUPS = B_PER_W // GROUP
NTILES = NROWS // 128
LAST_BASE = NROWS - 128
SAFE_LIMIT = NTILES * 128


def _rsqrt_newton(x):
    i = lax.bitcast_convert_type(x, jnp.int32)
    i = jnp.int32(0x5F3759DF) - (i >> 1)
    y = lax.bitcast_convert_type(i, jnp.float32)
    for _ in range(4):
        y = y * (jnp.float32(1.5) - jnp.float32(0.5) * x * y * y)
    return y


def _lane_pick(vec, lane_vec):
    return lax.gather(
        vec, lane_vec[:, None],
        dimension_numbers=lax.GatherDimensionNumbers(
            offset_dims=(), collapsed_slice_dims=(0,), start_index_map=(0,)),
        slice_sizes=(1,),
        mode=lax.GatherScatterMode.PROMISE_IN_BOUNDS)


_J16 = tuple(range(DIM))


def _extract_row(slot, last_v, safe_f, chunk, chunk_last, lane_vec):
    vals = []
    for j in _J16:
        a = slot[j, pl.ds(chunk, 16)]
        b = last_v[j, pl.ds(chunk_last, 16)]
        ch = b + (a - b) * safe_f
        vals.append(_lane_pick(ch, lane_vec))
    return vals


def _gather_body(uidx_hbm, iidx_hbm, utab_hbm, itab_hbm, ulast_hbm,
                 ilast_hbm, y_hbm, part_hbm,
                 uidx_v, iidx_v, slot_u, slot_i, last_u, last_i, yv,
                 part_v, sem):
    wid = lax.axis_index("s") * 2 + lax.axis_index("c")
    base = wid * B_PER_W

    pltpu.sync_copy(uidx_hbm.at[pl.ds(base, B_PER_W)], uidx_v)
    pltpu.sync_copy(iidx_hbm.at[pl.ds(base, B_PER_W)], iidx_v)
    pltpu.sync_copy(ulast_hbm, last_u)
    pltpu.sync_copy(ilast_hbm, last_i)

    iota16 = lax.iota(jnp.int32, 16)
    zeros = jnp.zeros((16,), jnp.float32)
    onehot = [
        (1 - jnp.minimum(jnp.abs(iota16 - j), 1)).astype(jnp.float32)
        for j in _J16
    ]

    def fire(uvec, ivec, k):
        ur = uvec[k]
        ir = ivec[k]
        uc = jnp.minimum(ur >> 7, NTILES - 1)
        ic = jnp.minimum(ir >> 7, NTILES - 1)
        ucol = pl.multiple_of(uc << 7, 128)
        icol = pl.multiple_of(ic << 7, 128)
        pltpu.async_copy(utab_hbm.at[:, pl.ds(ucol, 128)],
                         slot_u.at[k], sem.at[k])
        pltpu.async_copy(itab_hbm.at[:, pl.ds(icol, 128)],
                         slot_i.at[k], sem.at[k])

    uvec0 = uidx_v[pl.ds(0, GROUP)]
    ivec0 = iidx_v[pl.ds(0, GROUP)]
    for k in range(GROUP):
        fire(uvec0, ivec0, k)

    def group(g, carry):
        su, sv = carry
        uvec = uidx_v[pl.ds(g * GROUP, GROUP)]
        ivec = iidx_v[pl.ds(g * GROUP, GROUP)]
        gn = jnp.minimum(g + 1, NGROUPS - 1)
        nuvec = uidx_v[pl.ds(gn * GROUP, GROUP)]
        nivec = iidx_v[pl.ds(gn * GROUP, GROUP)]

        for k in range(GROUP):
            pltpu.make_async_copy(utab_hbm.at[:, pl.ds(0, 128)],
                                  slot_u.at[k], sem.at[k]).wait()
            pltpu.make_async_copy(itab_hbm.at[:, pl.ds(0, 128)],
                                  slot_i.at[k], sem.at[k]).wait()
            ur = uvec[k]
            ir = ivec[k]

            u_safe = ur < SAFE_LIMIT
            i_safe = ir < SAFE_LIMIT
            us = ur & 127
            is_ = ir & 127
            usl = (ur - LAST_BASE) & 127
            isl = (ir - LAST_BASE) & 127
            u_vals = _extract_row(
                slot_u.at[k], last_u,
                jnp.broadcast_to(
                    jnp.where(u_safe, jnp.float32(1.0), jnp.float32(0.0)),
                    (16,)),
                pl.multiple_of((us >> 4) << 4, 16),
                pl.multiple_of((usl >> 4) << 4, 16),
                jnp.broadcast_to(jnp.where(u_safe, us, usl) & 15, (16,)))
            i_vals = _extract_row(
                slot_i.at[k], last_i,
                jnp.broadcast_to(
                    jnp.where(i_safe, jnp.float32(1.0), jnp.float32(0.0)),
                    (16,)),
                pl.multiple_of((is_ >> 4) << 4, 16),
                pl.multiple_of((isl >> 4) << 4, 16),
                jnp.broadcast_to(jnp.where(i_safe, is_, isl) & 15, (16,)))

            p = zeros
            for j in _J16:
                u = u_vals[j]
                v = i_vals[j]
                su = su + u * u
                sv = sv + v * v
                p = p + (u * v) * onehot[j]
            yv[pl.ds((g * GROUP + k) * DIM, DIM)] = p

            @pl.when(g + 1 < NGROUPS)
            def _():
                fire(nuvec, nivec, k)

        return su, sv

    su, sv = lax.fori_loop(0, NGROUPS, group, (zeros, zeros))

    part_v[pl.ds(0, 16)] = su
    part_v[pl.ds(16, 16)] = sv
    pltpu.sync_copy(part_v, part_hbm.at[pl.ds(wid * 32, 32)])
    pltpu.sync_copy(yv, y_hbm.at[pl.ds(base * DIM, B_PER_W * DIM)])


def _scale_body(y_hbm, part_hbm, out_hbm, yv, part_v):
    wid = lax.axis_index("s") * 2 + lax.axis_index("c")
    base = wid * B_PER_W * DIM

    pltpu.sync_copy(part_hbm, part_v)
    pltpu.sync_copy(y_hbm.at[pl.ds(base, B_PER_W * DIM)], yv)

    su = part_v[pl.ds(0, 16)]
    sv = part_v[pl.ds(16, 16)]
    for w in range(1, NW):
        su = su + part_v[pl.ds(w * 32, 16)]
        sv = sv + part_v[pl.ds(w * 32 + 16, 16)]
    inv = _rsqrt_newton(su * sv)

    def scale(i, _):
        yv[pl.ds(i * 16, 16)] = yv[pl.ds(i * 16, 16)] * inv
        return 0

    lax.fori_loop(0, B_PER_W * DIM // 16, scale, 0)
    pltpu.sync_copy(yv, out_hbm.at[pl.ds(base, B_PER_W * DIM)])


@jax.jit
def kernel(user_indices, item_indices, user_table, item_table):
    uidx = user_indices.astype(jnp.int32)
    iidx = item_indices.astype(jnp.int32)
    ut = user_table.T
    it = item_table.T
    ulast = user_table[LAST_BASE:].T
    ilast = item_table[LAST_BASE:].T

    mesh = plsc.VectorSubcoreMesh(
        core_axis_name="c", subcore_axis_name="s", num_cores=2,
        num_subcores=16)
    gather = pl.kernel(
        _gather_body,
        out_type=(
            jax.ShapeDtypeStruct((BATCH * DIM,), jnp.float32),
            jax.ShapeDtypeStruct((NW * 32,), jnp.float32),
        ),
        mesh=mesh,
        scratch_types=[
            pltpu.VMEM((B_PER_W,), jnp.int32),
            pltpu.VMEM((B_PER_W,), jnp.int32),
            pltpu.VMEM((GROUP, 16, 128), jnp.float32),
            pltpu.VMEM((GROUP, 16, 128), jnp.float32),
            pltpu.VMEM((16, 128), jnp.float32),
            pltpu.VMEM((16, 128), jnp.float32),
            pltpu.VMEM((B_PER_W * DIM,), jnp.float32),
            pltpu.VMEM((32,), jnp.float32),
            pltpu.SemaphoreType.DMA((GROUP,)),
        ],
        compiler_params=pltpu.CompilerParams(use_tc_tiling_on_sc=True),
    )
    y_un, partials = gather(uidx, iidx, ut, it, ulast, ilast)

    parts = partials.reshape(NW, 2, 16)
    inv = lax.rsqrt(jnp.sum(parts[:, 0, 0]) * jnp.sum(parts[:, 1, 0]))
    return (y_un * inv).reshape(BATCH, DIM)


# speedup vs baseline: 5.1705x; 1.0022x over previous
import jax
import jax.numpy as jnp
from jax import lax
from jax.experimental import pallas as pl
from jax.experimental.pallas import tpu as pltpu
from jax.experimental.pallas import tpu_sc as plsc

NROWS = 1_000_000
BATCH = 16384
DIM = 16
NW = 32
B_PER_W = BATCH // NW
GROUP = 16
NGROUPS = B_PER_W // GROUP
NTILES = NROWS // 128
LAST_BASE = NROWS - 128
SAFE_LIMIT = NTILES * 128


def _lane_pick(vec, lane_vec):
    return lax.gather(
        vec, lane_vec[:, None],
        dimension_numbers=lax.GatherDimensionNumbers(
            offset_dims=(), collapsed_slice_dims=(0,), start_index_map=(0,)),
        slice_sizes=(1,),
        mode=lax.GatherScatterMode.PROMISE_IN_BOUNDS)


_J16 = tuple(range(DIM))


def _extract_row(slot, last_v, safe_f, chunk, chunk_last, lane_vec):
    vals = []
    for j in _J16:
        a = slot[j, pl.ds(chunk, 16)]
        b = last_v[j, pl.ds(chunk_last, 16)]
        ch = b + (a - b) * safe_f
        vals.append(_lane_pick(ch, lane_vec))
    return vals


def _gather_body(uidx_hbm, iidx_hbm, utab_hbm, itab_hbm, ulast_hbm,
                 ilast_hbm, y_hbm, part_hbm,
                 uidx_v, iidx_v, slot_u, slot_i, last_u, last_i, yv,
                 part_v, sem):
    wid = lax.axis_index("s") * 2 + lax.axis_index("c")
    base = wid * B_PER_W

    pltpu.sync_copy(uidx_hbm.at[pl.ds(base, B_PER_W)], uidx_v)
    pltpu.sync_copy(iidx_hbm.at[pl.ds(base, B_PER_W)], iidx_v)
    pltpu.sync_copy(ulast_hbm, last_u)
    pltpu.sync_copy(ilast_hbm, last_i)

    iota16 = lax.iota(jnp.int32, 16)
    zeros = jnp.zeros((16,), jnp.float32)
    onehot = [
        (1 - jnp.minimum(jnp.abs(iota16 - j), 1)).astype(jnp.float32)
        for j in _J16
    ]

    def fire(uvec, ivec, k):
        ur = uvec[k]
        ir = ivec[k]
        uc = jnp.minimum(ur >> 7, NTILES - 1)
        ic = jnp.minimum(ir >> 7, NTILES - 1)
        ucol = pl.multiple_of(uc << 7, 128)
        icol = pl.multiple_of(ic << 7, 128)
        pltpu.async_copy(utab_hbm.at[:, pl.ds(ucol, 128)],
                         slot_u.at[k], sem.at[k])
        pltpu.async_copy(itab_hbm.at[:, pl.ds(icol, 128)],
                         slot_i.at[k], sem.at[k])

    uvec0 = uidx_v[pl.ds(0, GROUP)]
    ivec0 = iidx_v[pl.ds(0, GROUP)]
    for k in range(GROUP):
        fire(uvec0, ivec0, k)

    def group(g, carry):
        su, sv = carry
        uvec = uidx_v[pl.ds(g * GROUP, GROUP)]
        ivec = iidx_v[pl.ds(g * GROUP, GROUP)]
        gn = jnp.minimum(g + 1, NGROUPS - 1)
        nuvec = uidx_v[pl.ds(gn * GROUP, GROUP)]
        nivec = iidx_v[pl.ds(gn * GROUP, GROUP)]

        for k in range(GROUP):
            pltpu.make_async_copy(utab_hbm.at[:, pl.ds(0, 128)],
                                  slot_u.at[k], sem.at[k]).wait()
            pltpu.make_async_copy(itab_hbm.at[:, pl.ds(0, 128)],
                                  slot_i.at[k], sem.at[k]).wait()
            ur = uvec[k]
            ir = ivec[k]

            u_safe = ur < SAFE_LIMIT
            i_safe = ir < SAFE_LIMIT
            us = ur & 127
            is_ = ir & 127
            usl = (ur - LAST_BASE) & 127
            isl = (ir - LAST_BASE) & 127
            u_vals = _extract_row(
                slot_u.at[k], last_u,
                jnp.broadcast_to(
                    jnp.where(u_safe, jnp.float32(1.0), jnp.float32(0.0)),
                    (16,)),
                pl.multiple_of((us >> 4) << 4, 16),
                pl.multiple_of((usl >> 4) << 4, 16),
                jnp.broadcast_to(jnp.where(u_safe, us, usl) & 15, (16,)))
            i_vals = _extract_row(
                slot_i.at[k], last_i,
                jnp.broadcast_to(
                    jnp.where(i_safe, jnp.float32(1.0), jnp.float32(0.0)),
                    (16,)),
                pl.multiple_of((is_ >> 4) << 4, 16),
                pl.multiple_of((isl >> 4) << 4, 16),
                jnp.broadcast_to(jnp.where(i_safe, is_, isl) & 15, (16,)))

            p = zeros
            for j in _J16:
                u = u_vals[j]
                v = i_vals[j]
                su = su + u * u
                sv = sv + v * v
                p = p + (u * v) * onehot[j]
            yv[pl.ds((g * GROUP + k) * DIM, DIM)] = p

            @pl.when(g + 1 < NGROUPS)
            def _():
                fire(nuvec, nivec, k)

        return su, sv

    su, sv = lax.fori_loop(0, NGROUPS, group, (zeros, zeros))

    part_v[pl.ds(0, 16)] = su
    part_v[pl.ds(16, 16)] = sv
    pltpu.sync_copy(part_v, part_hbm.at[pl.ds(wid * 32, 32)])
    pltpu.sync_copy(yv, y_hbm.at[pl.ds(base * DIM, B_PER_W * DIM)])


@jax.jit
def kernel(user_indices, item_indices, user_table, item_table):
    uidx = user_indices.astype(jnp.int32)
    iidx = item_indices.astype(jnp.int32)
    ut = user_table.T
    it = item_table.T
    ulast = user_table[LAST_BASE:].T
    ilast = item_table[LAST_BASE:].T

    mesh = plsc.VectorSubcoreMesh(
        core_axis_name="c", subcore_axis_name="s", num_cores=2,
        num_subcores=16)
    gather = pl.kernel(
        _gather_body,
        out_type=(
            jax.ShapeDtypeStruct((BATCH * DIM,), jnp.float32),
            jax.ShapeDtypeStruct((NW * 32,), jnp.float32),
        ),
        mesh=mesh,
        scratch_types=[
            pltpu.VMEM((B_PER_W,), jnp.int32),
            pltpu.VMEM((B_PER_W,), jnp.int32),
            pltpu.VMEM((GROUP, 16, 128), jnp.float32),
            pltpu.VMEM((GROUP, 16, 128), jnp.float32),
            pltpu.VMEM((16, 128), jnp.float32),
            pltpu.VMEM((16, 128), jnp.float32),
            pltpu.VMEM((B_PER_W * DIM,), jnp.float32),
            pltpu.VMEM((32,), jnp.float32),
            pltpu.SemaphoreType.DMA((GROUP,)),
        ],
        compiler_params=pltpu.CompilerParams(use_tc_tiling_on_sc=True),
    )
    y_un, partials = gather(uidx, iidx, ut, it, ulast, ilast)

    parts = partials.reshape(NW, 2, 16)
    inv = lax.rsqrt(jnp.sum(parts[:, 0, 0]) * jnp.sum(parts[:, 1, 0]))
    return (y_un * inv).reshape(BATCH, DIM)
